# baseline (speedup 1.0000x reference)
import functools

import jax
import jax.numpy as jnp
from jax.experimental import pallas as pl
from jax.experimental.pallas import tpu as pltpu

_THR = 0.05
_LANES = 128
_NEWTON_ITERS = 24
_CHUNK = 8


def _stats_kernel(pp_ref, pts_ref, out_ref, acc_ref, *, nsteps, nplanes):
    step = pl.program_id(0)

    @pl.when(step == 0)
    def _init():
        acc_ref[...] = jnp.zeros_like(acc_ref)

    blk = pts_ref[...]
    x = blk[0, 0]
    y = blk[1, 0]
    z = blk[2, 0]
    xx = x * x
    xy = x * y
    xz = x * z
    yy = y * y
    yz = y * z
    zz = z * z
    one = jnp.ones_like(x)
    feats = (one, x, y, z, xx, xy, xz, yy, yz, zz)
    for i in range(nplanes):
        n0 = pp_ref[i, 0]
        n1 = pp_ref[i, 1]
        n2 = pp_ref[i, 2]
        d = pp_ref[i, 3]
        t = x * n0 + y * n1 + z * n2 - d
        m = jnp.where(jnp.abs(t) < _THR, 1.0, 0.0)
        for j, f in enumerate(feats):
            acc_ref[j, i, :] = acc_ref[j, i, :] + jnp.sum(m * f, axis=0)

    @pl.when(step == nsteps - 1)
    def _fin():
        out_ref[...] = acc_ref[...]


def _solve_kernel(stats_ref, ppv_ref, out_ref):
    S = jnp.sum(stats_ref[...], axis=-1)
    cnt = S[0]
    sx = S[1]
    sy = S[2]
    sz = S[3]
    denom = jnp.maximum(cnt, 1.0)
    inv_d = 1.0 / denom
    cx = sx * inv_d
    cy = sy * inv_d
    cz = sz * inv_d
    a00 = S[4] - sx * sx * inv_d
    a01 = S[5] - sx * sy * inv_d
    a02 = S[6] - sx * sz * inv_d
    a11 = S[7] - sy * sy * inv_d
    a12 = S[8] - sy * sz * inv_d
    a22 = S[9] - sz * sz * inv_d
    sc = jnp.maximum((a00 + a11 + a22) * (1.0 / 3.0), 1e-30)
    inv_sc = 1.0 / sc
    a00 = a00 * inv_sc
    a01 = a01 * inv_sc
    a02 = a02 * inv_sc
    a11 = a11 * inv_sc
    a12 = a12 * inv_sc
    a22 = a22 * inv_sc
    tr = a00 + a11 + a22
    e2 = (a00 * a11 - a01 * a01) + (a00 * a22 - a02 * a02) + (a11 * a22 - a12 * a12)
    det = (
        a00 * (a11 * a22 - a12 * a12)
        - a01 * (a01 * a22 - a12 * a02)
        + a02 * (a01 * a12 - a11 * a02)
    )
    lam = jnp.zeros_like(tr)
    for _ in range(_NEWTON_ITERS):
        pval = det + lam * (-e2 + lam * (tr - lam))
        pder = -3.0 * lam * lam + 2.0 * tr * lam - e2
        pder = jnp.minimum(pder, -1e-20)
        lam = lam - pval / pder
    s12 = tr - lam
    l1l2 = e2 - lam * s12
    b00 = a00 * a00 + a01 * a01 + a02 * a02
    b01 = a00 * a01 + a01 * a11 + a02 * a12
    b02 = a00 * a02 + a01 * a12 + a02 * a22
    b11 = a01 * a01 + a11 * a11 + a12 * a12
    b12 = a01 * a02 + a11 * a12 + a12 * a22
    b22 = a02 * a02 + a12 * a12 + a22 * a22
    c00 = b00 - s12 * a00 + l1l2
    c01 = b01 - s12 * a01
    c02 = b02 - s12 * a02
    c11 = b11 - s12 * a11 + l1l2
    c12 = b12 - s12 * a12
    c22 = b22 - s12 * a22 + l1l2
    q0 = c00 * c00 + c01 * c01 + c02 * c02
    q1 = c01 * c01 + c11 * c11 + c12 * c12
    q2 = c02 * c02 + c12 * c12 + c22 * c22
    use0 = (q0 >= q1) & (q0 >= q2)
    use1 = jnp.logical_not(use0) & (q1 >= q2)
    vx = jnp.where(use0, c00, jnp.where(use1, c01, c02))
    vy = jnp.where(use0, c01, jnp.where(use1, c11, c12))
    vz = jnp.where(use0, c02, jnp.where(use1, c12, c22))
    inv_n = jax.lax.rsqrt(jnp.maximum(vx * vx + vy * vy + vz * vz, 1e-38))
    vx = vx * inv_n
    vy = vy * inv_n
    vz = vz * inv_n
    n0 = ppv_ref[0, :]
    n1 = ppv_ref[1, :]
    n2 = ppv_ref[2, :]
    d = ppv_ref[3, :]
    sgn = jnp.where(vx * n0 + vy * n1 + vz * n2 < 0.0, -1.0, 1.0)
    vx = vx * sgn
    vy = vy * sgn
    vz = vz * sgn
    dr = cx * vx + cy * vy + cz * vz
    thr_eff = jnp.where(cnt >= 2.5, _THR, -1.0)
    out_ref[0, :] = n0
    out_ref[1, :] = n1
    out_ref[2, :] = n2
    out_ref[3, :] = d
    out_ref[4, :] = vx
    out_ref[5, :] = vy
    out_ref[6, :] = vz
    out_ref[7, :] = dr
    out_ref[8, :] = thr_eff


def _proj_kernel(rp_ref, pts_ref, out_ref, *, nplanes):
    blk = pts_ref[...]
    x0 = blk[0, 0]
    y0 = blk[1, 0]
    z0 = blk[2, 0]
    px = x0
    py = y0
    pz = z0
    for i in range(nplanes):
        n0 = rp_ref[0, i]
        n1 = rp_ref[1, i]
        n2 = rp_ref[2, i]
        d = rp_ref[3, i]
        r0 = rp_ref[4, i]
        r1 = rp_ref[5, i]
        r2 = rp_ref[6, i]
        dr = rp_ref[7, i]
        th = rp_ref[8, i]
        t = x0 * n0 + y0 * n1 + z0 * n2 - d
        mf = jnp.where(jnp.abs(t) < th, 1.0, 0.0)
        s = (px * r0 + py * r1 + pz * r2 - dr) * mf
        px = px - r0 * s
        py = py - r1 * s
        pz = pz - r2 * s
    out_ref[0, 0] = px
    out_ref[1, 0] = py
    out_ref[2, 0] = pz


def _pick_rows(rows):
    for cand in (3125, 625, 125, 25, 5, 1):
        if rows % cand == 0:
            return cand
    return rows


def kernel(points, normals, distances):
    n_pts = points.shape[0]
    nplanes = normals.shape[0]
    assert n_pts % _LANES == 0
    rows = n_pts // _LANES
    br = _pick_rows(rows)
    nsteps = rows // br

    pts_t = points.T.reshape(3, nsteps, br, _LANES)
    pp = jnp.concatenate([normals, distances[:, None]], axis=1)
    ppv = pp.T

    stats = pl.pallas_call(
        functools.partial(_stats_kernel, nsteps=nsteps, nplanes=nplanes),
        grid=(nsteps,),
        in_specs=[
            pl.BlockSpec(memory_space=pltpu.SMEM),
            pl.BlockSpec((3, 1, br, _LANES), lambda s: (0, s, 0, 0)),
        ],
        out_specs=pl.BlockSpec((10, nplanes, _LANES), lambda s: (0, 0, 0)),
        out_shape=jax.ShapeDtypeStruct((10, nplanes, _LANES), jnp.float32),
        scratch_shapes=[pltpu.VMEM((10, nplanes, _LANES), jnp.float32)],
    )(pp, pts_t)

    rp = pl.pallas_call(
        _solve_kernel,
        out_shape=jax.ShapeDtypeStruct((9, nplanes), jnp.float32),
    )(stats, ppv)

    out_t = pl.pallas_call(
        functools.partial(_proj_kernel, nplanes=nplanes),
        grid=(nsteps,),
        in_specs=[
            pl.BlockSpec(memory_space=pltpu.SMEM),
            pl.BlockSpec((3, 1, br, _LANES), lambda s: (0, s, 0, 0)),
        ],
        out_specs=pl.BlockSpec((3, 1, br, _LANES), lambda s: (0, s, 0, 0)),
        out_shape=jax.ShapeDtypeStruct((3, nsteps, br, _LANES), jnp.float32),
    )(rp, pts_t)

    return out_t.reshape(3, n_pts).T


# speedup vs baseline: 1.1067x; 1.1067x over previous
import functools

import jax
import jax.numpy as jnp
from jax.experimental import pallas as pl
from jax.experimental.pallas import tpu as pltpu

_THR = 0.05
_LANES = 128
_NEWTON_ITERS = 24
_CHUNK = 8


def _stats_kernel(pp_ref, ppv_ref, pts_ref, out_ref, acc_ref, *, nsteps, nplanes):
    step = pl.program_id(0)

    @pl.when(step == 0)
    def _init():
        acc_ref[...] = jnp.zeros_like(acc_ref)

    blk = pts_ref[...]
    x = blk[0, 0]
    y = blk[1, 0]
    z = blk[2, 0]
    xx = x * x
    xy = x * y
    xz = x * z
    yy = y * y
    yz = y * z
    zz = z * z
    one = jnp.ones_like(x)
    feats = (one, x, y, z, xx, xy, xz, yy, yz, zz)
    thr2 = _THR * _THR
    for i in range(nplanes):
        n0 = pp_ref[i, 0]
        n1 = pp_ref[i, 1]
        n2 = pp_ref[i, 2]
        d = pp_ref[i, 3]
        t = x * n0 + y * n1 + z * n2 - d
        m = jnp.where(t * t < thr2, 1.0, 0.0)
        for j, f in enumerate(feats):
            acc_ref[j, i, :] = acc_ref[j, i, :] + jnp.sum(m * f, axis=0)

    @pl.when(step == nsteps - 1)
    def _fin():
        _solve(acc_ref, ppv_ref, out_ref)


def _solve(stats_ref, ppv_ref, out_ref):
    S = jnp.sum(stats_ref[...], axis=-1)
    cnt = S[0]
    sx = S[1]
    sy = S[2]
    sz = S[3]
    denom = jnp.maximum(cnt, 1.0)
    inv_d = 1.0 / denom
    cx = sx * inv_d
    cy = sy * inv_d
    cz = sz * inv_d
    a00 = S[4] - sx * sx * inv_d
    a01 = S[5] - sx * sy * inv_d
    a02 = S[6] - sx * sz * inv_d
    a11 = S[7] - sy * sy * inv_d
    a12 = S[8] - sy * sz * inv_d
    a22 = S[9] - sz * sz * inv_d
    sc = jnp.maximum((a00 + a11 + a22) * (1.0 / 3.0), 1e-30)
    inv_sc = 1.0 / sc
    a00 = a00 * inv_sc
    a01 = a01 * inv_sc
    a02 = a02 * inv_sc
    a11 = a11 * inv_sc
    a12 = a12 * inv_sc
    a22 = a22 * inv_sc
    tr = a00 + a11 + a22
    e2 = (a00 * a11 - a01 * a01) + (a00 * a22 - a02 * a02) + (a11 * a22 - a12 * a12)
    det = (
        a00 * (a11 * a22 - a12 * a12)
        - a01 * (a01 * a22 - a12 * a02)
        + a02 * (a01 * a12 - a11 * a02)
    )
    lam = jnp.zeros_like(tr)
    for _ in range(_NEWTON_ITERS):
        pval = det + lam * (-e2 + lam * (tr - lam))
        pder = -3.0 * lam * lam + 2.0 * tr * lam - e2
        pder = jnp.minimum(pder, -1e-20)
        lam = lam - pval / pder
    s12 = tr - lam
    l1l2 = e2 - lam * s12
    b00 = a00 * a00 + a01 * a01 + a02 * a02
    b01 = a00 * a01 + a01 * a11 + a02 * a12
    b02 = a00 * a02 + a01 * a12 + a02 * a22
    b11 = a01 * a01 + a11 * a11 + a12 * a12
    b12 = a01 * a02 + a11 * a12 + a12 * a22
    b22 = a02 * a02 + a12 * a12 + a22 * a22
    c00 = b00 - s12 * a00 + l1l2
    c01 = b01 - s12 * a01
    c02 = b02 - s12 * a02
    c11 = b11 - s12 * a11 + l1l2
    c12 = b12 - s12 * a12
    c22 = b22 - s12 * a22 + l1l2
    q0 = c00 * c00 + c01 * c01 + c02 * c02
    q1 = c01 * c01 + c11 * c11 + c12 * c12
    q2 = c02 * c02 + c12 * c12 + c22 * c22
    use0 = (q0 >= q1) & (q0 >= q2)
    use1 = jnp.logical_not(use0) & (q1 >= q2)
    vx = jnp.where(use0, c00, jnp.where(use1, c01, c02))
    vy = jnp.where(use0, c01, jnp.where(use1, c11, c12))
    vz = jnp.where(use0, c02, jnp.where(use1, c12, c22))
    inv_n = jax.lax.rsqrt(jnp.maximum(vx * vx + vy * vy + vz * vz, 1e-38))
    vx = vx * inv_n
    vy = vy * inv_n
    vz = vz * inv_n
    n0 = ppv_ref[0, :]
    n1 = ppv_ref[1, :]
    n2 = ppv_ref[2, :]
    d = ppv_ref[3, :]
    sgn = jnp.where(vx * n0 + vy * n1 + vz * n2 < 0.0, -1.0, 1.0)
    vx = vx * sgn
    vy = vy * sgn
    vz = vz * sgn
    dr = cx * vx + cy * vy + cz * vz
    thr_eff = jnp.where(cnt >= 2.5, _THR * _THR, -1.0)
    out_ref[0, :] = n0
    out_ref[1, :] = n1
    out_ref[2, :] = n2
    out_ref[3, :] = d
    out_ref[4, :] = vx
    out_ref[5, :] = vy
    out_ref[6, :] = vz
    out_ref[7, :] = dr
    out_ref[8, :] = thr_eff


def _proj_kernel(rp_ref, pts_ref, out_ref, *, nplanes):
    blk = pts_ref[...]
    x0 = blk[0, 0]
    y0 = blk[1, 0]
    z0 = blk[2, 0]
    px = x0
    py = y0
    pz = z0
    for i in range(nplanes):
        n0 = rp_ref[0, i]
        n1 = rp_ref[1, i]
        n2 = rp_ref[2, i]
        d = rp_ref[3, i]
        r0 = rp_ref[4, i]
        r1 = rp_ref[5, i]
        r2 = rp_ref[6, i]
        dr = rp_ref[7, i]
        th = rp_ref[8, i]
        t = x0 * n0 + y0 * n1 + z0 * n2 - d
        mf = jnp.where(t * t < th, 1.0, 0.0)
        s = (px * r0 + py * r1 + pz * r2 - dr) * mf
        px = px - r0 * s
        py = py - r1 * s
        pz = pz - r2 * s
    out_ref[0, 0] = px
    out_ref[1, 0] = py
    out_ref[2, 0] = pz


def _pick_rows(rows):
    for cand in (625, 125, 25, 5, 1):
        if rows % cand == 0:
            return cand
    return rows


def kernel(points, normals, distances):
    n_pts = points.shape[0]
    nplanes = normals.shape[0]
    assert n_pts % _LANES == 0
    rows = n_pts // _LANES
    br = _pick_rows(rows)
    nsteps = rows // br

    pts_t = points.T.reshape(3, nsteps, br, _LANES)
    pp = jnp.concatenate([normals, distances[:, None]], axis=1)
    ppv = pp.T

    rp = pl.pallas_call(
        functools.partial(_stats_kernel, nsteps=nsteps, nplanes=nplanes),
        grid=(nsteps,),
        in_specs=[
            pl.BlockSpec(memory_space=pltpu.SMEM),
            pl.BlockSpec((4, nplanes), lambda s: (0, 0)),
            pl.BlockSpec((3, 1, br, _LANES), lambda s: (0, s, 0, 0)),
        ],
        out_specs=pl.BlockSpec((9, nplanes), lambda s: (0, 0)),
        out_shape=jax.ShapeDtypeStruct((9, nplanes), jnp.float32),
        scratch_shapes=[pltpu.VMEM((10, nplanes, _LANES), jnp.float32)],
    )(pp, ppv, pts_t)

    out_t = pl.pallas_call(
        functools.partial(_proj_kernel, nplanes=nplanes),
        grid=(nsteps,),
        in_specs=[
            pl.BlockSpec(memory_space=pltpu.SMEM),
            pl.BlockSpec((3, 1, br, _LANES), lambda s: (0, s, 0, 0)),
        ],
        out_specs=pl.BlockSpec((3, 1, br, _LANES), lambda s: (0, s, 0, 0)),
        out_shape=jax.ShapeDtypeStruct((3, nsteps, br, _LANES), jnp.float32),
    )(rp, pts_t)

    return out_t.reshape(3, n_pts).T


# speedup vs baseline: 1.1286x; 1.0198x over previous
import functools

import jax
import jax.numpy as jnp
from jax.experimental import pallas as pl
from jax.experimental.pallas import tpu as pltpu

_THR = 0.05
_LANES = 128
_NEWTON_ITERS = 24
_CHUNK = 8


def _stats_kernel(pp_ref, ppv_ref, pts_ref, out_ref, bits_ref, acc_ref, *, nsteps, nplanes):
    step = pl.program_id(0)

    @pl.when(step == 0)
    def _init():
        acc_ref[...] = jnp.zeros_like(acc_ref)

    blk = pts_ref[...]
    x = blk[0, 0]
    y = blk[1, 0]
    z = blk[2, 0]
    xx = x * x
    xy = x * y
    xz = x * z
    yy = y * y
    yz = y * z
    zz = z * z
    one = jnp.ones_like(x)
    feats = (one, x, y, z, xx, xy, xz, yy, yz, zz)
    thr2 = _THR * _THR
    bits = jnp.zeros(x.shape, jnp.int32)
    for i in range(nplanes):
        n0 = pp_ref[i, 0]
        n1 = pp_ref[i, 1]
        n2 = pp_ref[i, 2]
        d = pp_ref[i, 3]
        t = x * n0 + y * n1 + z * n2 - d
        hit = t * t < thr2
        m = jnp.where(hit, 1.0, 0.0)
        bits = bits | jnp.where(hit, jnp.int32(1 << i), jnp.int32(0))
        for j, f in enumerate(feats):
            acc_ref[j, i, :] = acc_ref[j, i, :] + jnp.sum(m * f, axis=0)
    bits_ref[0] = bits

    @pl.when(step == nsteps - 1)
    def _fin():
        _solve(acc_ref, ppv_ref, out_ref)


def _solve(stats_ref, ppv_ref, out_ref):
    S = jnp.sum(stats_ref[...], axis=-1)
    cnt = S[0]
    sx = S[1]
    sy = S[2]
    sz = S[3]
    denom = jnp.maximum(cnt, 1.0)
    inv_d = 1.0 / denom
    cx = sx * inv_d
    cy = sy * inv_d
    cz = sz * inv_d
    a00 = S[4] - sx * sx * inv_d
    a01 = S[5] - sx * sy * inv_d
    a02 = S[6] - sx * sz * inv_d
    a11 = S[7] - sy * sy * inv_d
    a12 = S[8] - sy * sz * inv_d
    a22 = S[9] - sz * sz * inv_d
    sc = jnp.maximum((a00 + a11 + a22) * (1.0 / 3.0), 1e-30)
    inv_sc = 1.0 / sc
    a00 = a00 * inv_sc
    a01 = a01 * inv_sc
    a02 = a02 * inv_sc
    a11 = a11 * inv_sc
    a12 = a12 * inv_sc
    a22 = a22 * inv_sc
    tr = a00 + a11 + a22
    e2 = (a00 * a11 - a01 * a01) + (a00 * a22 - a02 * a02) + (a11 * a22 - a12 * a12)
    det = (
        a00 * (a11 * a22 - a12 * a12)
        - a01 * (a01 * a22 - a12 * a02)
        + a02 * (a01 * a12 - a11 * a02)
    )
    lam = jnp.zeros_like(tr)
    for _ in range(_NEWTON_ITERS):
        pval = det + lam * (-e2 + lam * (tr - lam))
        pder = -3.0 * lam * lam + 2.0 * tr * lam - e2
        pder = jnp.minimum(pder, -1e-20)
        lam = lam - pval / pder
    s12 = tr - lam
    l1l2 = e2 - lam * s12
    b00 = a00 * a00 + a01 * a01 + a02 * a02
    b01 = a00 * a01 + a01 * a11 + a02 * a12
    b02 = a00 * a02 + a01 * a12 + a02 * a22
    b11 = a01 * a01 + a11 * a11 + a12 * a12
    b12 = a01 * a02 + a11 * a12 + a12 * a22
    b22 = a02 * a02 + a12 * a12 + a22 * a22
    c00 = b00 - s12 * a00 + l1l2
    c01 = b01 - s12 * a01
    c02 = b02 - s12 * a02
    c11 = b11 - s12 * a11 + l1l2
    c12 = b12 - s12 * a12
    c22 = b22 - s12 * a22 + l1l2
    q0 = c00 * c00 + c01 * c01 + c02 * c02
    q1 = c01 * c01 + c11 * c11 + c12 * c12
    q2 = c02 * c02 + c12 * c12 + c22 * c22
    use0 = (q0 >= q1) & (q0 >= q2)
    use1 = jnp.logical_not(use0) & (q1 >= q2)
    vx = jnp.where(use0, c00, jnp.where(use1, c01, c02))
    vy = jnp.where(use0, c01, jnp.where(use1, c11, c12))
    vz = jnp.where(use0, c02, jnp.where(use1, c12, c22))
    inv_n = jax.lax.rsqrt(jnp.maximum(vx * vx + vy * vy + vz * vz, 1e-38))
    vx = vx * inv_n
    vy = vy * inv_n
    vz = vz * inv_n
    n0 = ppv_ref[0, :]
    n1 = ppv_ref[1, :]
    n2 = ppv_ref[2, :]
    d = ppv_ref[3, :]
    sgn = jnp.where(vx * n0 + vy * n1 + vz * n2 < 0.0, -1.0, 1.0)
    vx = vx * sgn
    vy = vy * sgn
    vz = vz * sgn
    dr = cx * vx + cy * vy + cz * vz
    thr_eff = jnp.where(cnt >= 2.5, _THR * _THR, -1.0)
    out_ref[0, :] = n0
    out_ref[1, :] = n1
    out_ref[2, :] = n2
    out_ref[3, :] = d
    out_ref[4, :] = vx
    out_ref[5, :] = vy
    out_ref[6, :] = vz
    out_ref[7, :] = dr
    out_ref[8, :] = thr_eff


def _proj_kernel(rp_ref, pts_ref, bits_ref, out_ref, *, nplanes):
    blk = pts_ref[...]
    px = blk[0, 0]
    py = blk[1, 0]
    pz = blk[2, 0]
    bits = bits_ref[0]
    for i in range(nplanes):
        r0 = rp_ref[4, i]
        r1 = rp_ref[5, i]
        r2 = rp_ref[6, i]
        dr = rp_ref[7, i]
        th = rp_ref[8, i]
        gate = jnp.where(th > 0.0, 1.0, 0.0)
        mf = jnp.where((bits & jnp.int32(1 << i)) != 0, gate, 0.0)
        s = (px * r0 + py * r1 + pz * r2 - dr) * mf
        px = px - r0 * s
        py = py - r1 * s
        pz = pz - r2 * s
    out_ref[0, 0] = px
    out_ref[1, 0] = py
    out_ref[2, 0] = pz


def _pick_rows(rows):
    for cand in (625, 125, 25, 5, 1):
        if rows % cand == 0:
            return cand
    return rows


def kernel(points, normals, distances):
    n_pts = points.shape[0]
    nplanes = normals.shape[0]
    assert n_pts % _LANES == 0
    rows = n_pts // _LANES
    br = _pick_rows(rows)
    nsteps = rows // br

    pts_t = points.T.reshape(3, nsteps, br, _LANES)
    pp = jnp.concatenate([normals, distances[:, None]], axis=1)
    ppv = pp.T

    rp, bits = pl.pallas_call(
        functools.partial(_stats_kernel, nsteps=nsteps, nplanes=nplanes),
        grid=(nsteps,),
        in_specs=[
            pl.BlockSpec(memory_space=pltpu.SMEM),
            pl.BlockSpec((4, nplanes), lambda s: (0, 0)),
            pl.BlockSpec((3, 1, br, _LANES), lambda s: (0, s, 0, 0)),
        ],
        out_specs=[
            pl.BlockSpec((9, nplanes), lambda s: (0, 0)),
            pl.BlockSpec((1, br, _LANES), lambda s: (s, 0, 0)),
        ],
        out_shape=[
            jax.ShapeDtypeStruct((9, nplanes), jnp.float32),
            jax.ShapeDtypeStruct((nsteps, br, _LANES), jnp.int32),
        ],
        scratch_shapes=[pltpu.VMEM((10, nplanes, _LANES), jnp.float32)],
    )(pp, ppv, pts_t)

    out_t = pl.pallas_call(
        functools.partial(_proj_kernel, nplanes=nplanes),
        grid=(nsteps,),
        in_specs=[
            pl.BlockSpec(memory_space=pltpu.SMEM),
            pl.BlockSpec((3, 1, br, _LANES), lambda s: (0, s, 0, 0)),
            pl.BlockSpec((1, br, _LANES), lambda s: (s, 0, 0)),
        ],
        out_specs=pl.BlockSpec((3, 1, br, _LANES), lambda s: (0, s, 0, 0)),
        out_shape=jax.ShapeDtypeStruct((3, nsteps, br, _LANES), jnp.float32),
    )(rp, pts_t, bits)

    return out_t.reshape(3, n_pts).T
